# Initial kernel scaffold; baseline (speedup 1.0000x reference)
import jax
import jax.numpy as jnp
from jax.experimental import pallas as pl


def kernel(node_input, node_attr, edge_src, edge_dst, edge_attr, edge_scalars, W_sc, W_lin1, W_lin2, W_lin3, fc_w0, fc_w1):
    raise NotImplementedError("write your pallas kernel here")


# speedup vs baseline: 2.3166x; 2.3166x over previous
import functools

import jax
import jax.numpy as jnp
from jax import lax
from jax.experimental import pallas as pl
from jax.experimental.pallas import tpu as pltpu
from jax.experimental.pallas import tpu_sc as plsc

N_NODES = 10000
N_EDGES = 320000
D = 128
FC_IN = 16
FC_HID = 64
SILU_GAIN = 1.6789
RS_D = 1.0 / (D ** 0.5)
RS_IN = 1.0 / (FC_IN ** 0.5)
RS_HID = 1.0 / (FC_HID ** 0.5)
RS_NEI = 1.0 / (32.0 ** 0.5)

NC = 2
NS = 16
NW = NC * NS
EPW = N_EDGES // NW
K = 80
NCHUNK = EPW // K
N_PAD = 10240
ROWS_PER_SUB = N_PAD // NS


def _node_mm_body(x_ref, attr_ref, wsc_ref, wl1_ref, nsc_ref, nf_ref):
    t = x_ref[...] * attr_ref[...]
    nsc_ref[...] = jnp.dot(t, wsc_ref[...], preferred_element_type=jnp.float32) * RS_D
    nf_ref[...] = jnp.dot(t, wl1_ref[...], preferred_element_type=jnp.float32) * RS_D


def _node_mm(x, attr, W_sc, W_lin1, bn=2000):
    grid = (N_NODES // bn,)
    return pl.pallas_call(
        _node_mm_body,
        grid=grid,
        in_specs=[
            pl.BlockSpec((bn, D), lambda i: (i, 0)),
            pl.BlockSpec((bn, 1), lambda i: (i, 0)),
            pl.BlockSpec((D, D), lambda i: (0, 0)),
            pl.BlockSpec((D, D), lambda i: (0, 0)),
        ],
        out_specs=[
            pl.BlockSpec((bn, D), lambda i: (i, 0)),
            pl.BlockSpec((bn, D), lambda i: (i, 0)),
        ],
        out_shape=[
            jax.ShapeDtypeStruct((N_NODES, D), jnp.float32),
            jax.ShapeDtypeStruct((N_NODES, D), jnp.float32),
        ],
    )(x, attr, W_sc, W_lin1)


def _edge_mlp_body(es_ref, ea_ref, w0_ref, w1_ref, coeff_ref):
    h = jnp.dot(es_ref[...], w0_ref[...], preferred_element_type=jnp.float32) * RS_IN
    h = jax.nn.silu(h) * SILU_GAIN
    w = jnp.dot(h, w1_ref[...], preferred_element_type=jnp.float32) * RS_HID
    coeff_ref[...] = w * ea_ref[...]


def _edge_mlp(edge_scalars, edge_attr, fc_w0, fc_w1, be=4000):
    grid = (N_EDGES // be,)
    return pl.pallas_call(
        _edge_mlp_body,
        grid=grid,
        in_specs=[
            pl.BlockSpec((be, FC_IN), lambda i: (i, 0)),
            pl.BlockSpec((be, 1), lambda i: (i, 0)),
            pl.BlockSpec((FC_IN, FC_HID), lambda i: (0, 0)),
            pl.BlockSpec((FC_HID, D), lambda i: (0, 0)),
        ],
        out_specs=pl.BlockSpec((be, D), lambda i: (i, 0)),
        out_shape=jax.ShapeDtypeStruct((N_EDGES, D), jnp.float32),
    )(edge_scalars, edge_attr, fc_w0, fc_w1)


def _sc_gather_scatter(nf, coeff, src, dst):
    mesh = plsc.VectorSubcoreMesh(core_axis_name="c", subcore_axis_name="s")

    @functools.partial(
        pl.kernel,
        mesh=mesh,
        out_type=jax.ShapeDtypeStruct((NC * N_PAD, D), jnp.float32),
        scratch_types=[
            pltpu.VMEM((K,), jnp.int32),
            pltpu.VMEM((K,), jnp.int32),
            pltpu.VMEM((K, D), jnp.float32),
            pltpu.VMEM((K, D), jnp.float32),
            pltpu.VMEM_SHARED((N_PAD, D), jnp.float32),
            pltpu.SemaphoreType.DMA,
        ],
    )
    def body(nf_hbm, coeff_hbm, src_hbm, dst_hbm, out_hbm,
             sidx_v, didx_v, rows_v, coef_v, acc, sem):
        cid = lax.axis_index("c")
        sid = lax.axis_index("s")
        wid = sid * NC + cid

        zrow = sid * ROWS_PER_SUB
        rows_v[...] = jnp.zeros((K, D), jnp.float32)
        nfull = ROWS_PER_SUB // K
        for t in range(nfull):
            pltpu.sync_copy(rows_v, acc.at[pl.ds(zrow + t * K, K)])
        plsc.subcore_barrier()

        def chunk(i, carry):
            base = wid * EPW + i * K
            pltpu.sync_copy(src_hbm.at[pl.ds(base, K)], sidx_v)
            pltpu.sync_copy(dst_hbm.at[pl.ds(base, K)], didx_v)
            pltpu.sync_copy(coeff_hbm.at[pl.ds(base, K)], coef_v)
            pltpu.async_copy(nf_hbm.at[sidx_v], rows_v, sem).wait()

            def mul_row(k, c2):
                for j in range(D // 16):
                    s = pl.ds(j * 16, 16)
                    rows_v[k, s] = rows_v[k, s] * coef_v[k, s]
                return c2
            lax.fori_loop(0, K, mul_row, 0)

            pltpu.sync_copy(rows_v, acc.at[didx_v], add=True)
            return carry
        lax.fori_loop(0, NCHUNK, chunk, 0)

        plsc.subcore_barrier()

        orow = cid * N_PAD + zrow
        for t in range(nfull):
            pltpu.sync_copy(acc.at[pl.ds(zrow + t * K, K)], rows_v)
            pltpu.sync_copy(rows_v, out_hbm.at[pl.ds(orow + t * K, K)])

    return body(nf, coeff, src, dst)


def _final_body(p_ref, nsc_ref, attr_ref, wl2_ref, wl3_ref, out_ref):
    nf2 = (p_ref[0] + p_ref[1]) * RS_NEI
    t = nf2 * attr_ref[...]
    conv = jnp.dot(t, wl2_ref[...], preferred_element_type=jnp.float32) * RS_D
    ang = jnp.dot(t, wl3_ref[...], preferred_element_type=jnp.float32) * (0.1 * RS_D)
    out_ref[...] = jnp.cos(ang) * nsc_ref[...] + jnp.sin(ang) * conv


def _final(partials, nsc, attr, W_lin2, W_lin3, bn=2000):
    grid = (N_NODES // bn,)
    return pl.pallas_call(
        _final_body,
        grid=grid,
        in_specs=[
            pl.BlockSpec((2, bn, D), lambda i: (0, i, 0)),
            pl.BlockSpec((bn, D), lambda i: (i, 0)),
            pl.BlockSpec((bn, 1), lambda i: (i, 0)),
            pl.BlockSpec((D, D), lambda i: (0, 0)),
            pl.BlockSpec((D, 1), lambda i: (0, 0)),
        ],
        out_specs=pl.BlockSpec((bn, D), lambda i: (i, 0)),
        out_shape=jax.ShapeDtypeStruct((N_NODES, D), jnp.float32),
    )(partials, nsc, attr, W_lin2, W_lin3)


def kernel(node_input, node_attr, edge_src, edge_dst, edge_attr, edge_scalars,
           W_sc, W_lin1, W_lin2, W_lin3, fc_w0, fc_w1):
    src = edge_src.astype(jnp.int32)
    dst = edge_dst.astype(jnp.int32)
    nsc, nf = _node_mm(node_input, node_attr, W_sc, W_lin1)
    coeff = _edge_mlp(edge_scalars, edge_attr, fc_w0, fc_w1)
    partials = _sc_gather_scatter(nf, coeff, src, dst)
    partials = partials.reshape(NC, N_PAD, D)
    return _final(partials, nsc, node_attr, W_lin2, W_lin3)


# speedup vs baseline: 3.2193x; 1.3897x over previous
import functools

import jax
import jax.numpy as jnp
from jax import lax
from jax.experimental import pallas as pl
from jax.experimental.pallas import tpu as pltpu
from jax.experimental.pallas import tpu_sc as plsc

N_NODES = 10000
N_EDGES = 320000
D = 128
FC_IN = 16
FC_HID = 64
SILU_GAIN = 1.6789
RS_D = 1.0 / (D ** 0.5)
RS_IN = 1.0 / (FC_IN ** 0.5)
RS_HID = 1.0 / (FC_HID ** 0.5)
RS_NEI = 1.0 / (32.0 ** 0.5)

NC = 2
NS = 16
NW = NC * NS
EPW = N_EDGES // NW
K = 80
NCHUNK = EPW // K
N_PAD = 10240
ROWS_PER_SUB = N_PAD // NS


def _node_mm_body(x_ref, attr_ref, wsc_ref, wl1_ref, nsc_ref, nf_ref):
    t = x_ref[...] * attr_ref[...]
    nsc_ref[...] = jnp.dot(t, wsc_ref[...], preferred_element_type=jnp.float32) * RS_D
    nf_ref[...] = jnp.dot(t, wl1_ref[...], preferred_element_type=jnp.float32) * RS_D


def _node_mm(x, attr, W_sc, W_lin1, bn=2000):
    grid = (N_NODES // bn,)
    return pl.pallas_call(
        _node_mm_body,
        grid=grid,
        in_specs=[
            pl.BlockSpec((bn, D), lambda i: (i, 0)),
            pl.BlockSpec((bn, 1), lambda i: (i, 0)),
            pl.BlockSpec((D, D), lambda i: (0, 0)),
            pl.BlockSpec((D, D), lambda i: (0, 0)),
        ],
        out_specs=[
            pl.BlockSpec((bn, D), lambda i: (i, 0)),
            pl.BlockSpec((bn, D), lambda i: (i, 0)),
        ],
        out_shape=[
            jax.ShapeDtypeStruct((N_NODES, D), jnp.float32),
            jax.ShapeDtypeStruct((N_NODES, D), jnp.float32),
        ],
    )(x, attr, W_sc, W_lin1)


def _edge_mlp_body(es_ref, ea_ref, w0_ref, w1_ref, coeff_ref):
    h = jnp.dot(es_ref[...], w0_ref[...], preferred_element_type=jnp.float32) * RS_IN
    h = jax.nn.silu(h) * SILU_GAIN
    w = jnp.dot(h, w1_ref[...], preferred_element_type=jnp.float32) * RS_HID
    coeff_ref[...] = w * ea_ref[...]


def _edge_mlp(edge_scalars, edge_attr, fc_w0, fc_w1, be=4000):
    grid = (N_EDGES // be,)
    return pl.pallas_call(
        _edge_mlp_body,
        grid=grid,
        in_specs=[
            pl.BlockSpec((be, FC_IN), lambda i: (i, 0)),
            pl.BlockSpec((be, 1), lambda i: (i, 0)),
            pl.BlockSpec((FC_IN, FC_HID), lambda i: (0, 0)),
            pl.BlockSpec((FC_HID, D), lambda i: (0, 0)),
        ],
        out_specs=pl.BlockSpec((be, D), lambda i: (i, 0)),
        out_shape=jax.ShapeDtypeStruct((N_EDGES, D), jnp.float32),
    )(edge_scalars, edge_attr, fc_w0, fc_w1)


def _sc_gather_scatter(nf, coeff, src, dst):
    mesh = plsc.VectorSubcoreMesh(core_axis_name="c", subcore_axis_name="s")

    @functools.partial(
        pl.kernel,
        mesh=mesh,
        out_type=jax.ShapeDtypeStruct((NC * N_PAD, D), jnp.float32),
        scratch_types=[
            pltpu.VMEM((K,), jnp.int32),
            pltpu.VMEM((K,), jnp.int32),
            pltpu.VMEM((K, D), jnp.float32),
            pltpu.VMEM((K, D), jnp.float32),
            pltpu.VMEM((K,), jnp.int32),
            pltpu.VMEM((K,), jnp.int32),
            pltpu.VMEM((K, D), jnp.float32),
            pltpu.VMEM((K, D), jnp.float32),
            pltpu.VMEM_SHARED((N_PAD, D), jnp.float32),
            pltpu.SemaphoreType.DMA,
            pltpu.SemaphoreType.DMA,
            pltpu.SemaphoreType.DMA,
            pltpu.SemaphoreType.DMA,
            pltpu.SemaphoreType.DMA,
            pltpu.SemaphoreType.DMA,
        ],
    )
    def body(nf_hbm, coeff_hbm, src_hbm, dst_hbm, out_hbm,
             sidx0, didx0, rows0, coef0, sidx1, didx1, rows1, coef1,
             acc, isem0, isem1, gsem0, gsem1, csem0, csem1):
        cid = lax.axis_index("c")
        sid = lax.axis_index("s")
        wid = sid * NC + cid
        ebase = wid * EPW

        slot0 = (sidx0, didx0, rows0, coef0, isem0, gsem0, csem0)
        slot1 = (sidx1, didx1, rows1, coef1, isem1, gsem1, csem1)

        def issue_idx(ci, slot):
            sidx, didx, _, _, isem, _, _ = slot
            base = ebase + ci * K
            pltpu.async_copy(src_hbm.at[pl.ds(base, K)], sidx, isem)
            pltpu.async_copy(dst_hbm.at[pl.ds(base, K)], didx, isem)

        def wait_idx(slot):
            sidx, didx, _, _, isem, _, _ = slot
            pltpu.make_async_copy(src_hbm.at[pl.ds(0, K)], sidx, isem).wait()
            pltpu.make_async_copy(dst_hbm.at[pl.ds(0, K)], didx, isem).wait()

        def issue_gc(ci, slot):
            sidx, _, rows, coef, _, gsem, csem = slot
            pltpu.async_copy(nf_hbm.at[sidx], rows, gsem)
            pltpu.async_copy(coeff_hbm.at[pl.ds(ebase + ci * K, K)], coef, csem)

        def process(slot):
            _, didx, rows, coef, _, gsem, csem = slot
            pltpu.make_async_copy(nf_hbm.at[slot[0]], rows, gsem).wait()
            pltpu.make_async_copy(coeff_hbm.at[pl.ds(0, K)], coef, csem).wait()

            def mul_row(k, c2):
                for j in range(D // 16):
                    s = pl.ds(j * 16, 16)
                    rows[k, s] = rows[k, s] * coef[k, s]
                return c2
            lax.fori_loop(0, K, mul_row, 0)
            pltpu.sync_copy(rows, acc.at[didx], add=True)

        zrow = sid * ROWS_PER_SUB
        rows0[...] = jnp.zeros((K, D), jnp.float32)
        nfull = ROWS_PER_SUB // K
        for t in range(nfull):
            pltpu.sync_copy(rows0, acc.at[pl.ds(zrow + t * K, K)])
        plsc.subcore_barrier()

        issue_idx(0, slot0)
        wait_idx(slot0)
        issue_gc(0, slot0)
        issue_idx(1, slot1)

        def pair(t, carry):
            ca = 2 * t
            cb = 2 * t + 1
            wait_idx(slot1)
            issue_gc(cb, slot1)
            process(slot0)
            issue_idx(ca + 2, slot0)
            process(slot1)
            wait_idx(slot0)
            issue_gc(ca + 2, slot0)
            issue_idx(jnp.minimum(cb + 2, NCHUNK - 1), slot1)
            return carry
        lax.fori_loop(0, (NCHUNK - 1) // 2, pair, 0)

        wait_idx(slot1)
        process(slot0)

        plsc.subcore_barrier()

        orow = cid * N_PAD + zrow
        for t in range(nfull):
            pltpu.sync_copy(acc.at[pl.ds(zrow + t * K, K)], rows0)
            pltpu.sync_copy(rows0, out_hbm.at[pl.ds(orow + t * K, K)])

    return body(nf, coeff, src, dst)


def _final_body(p_ref, nsc_ref, attr_ref, wl2_ref, wl3_ref, out_ref):
    nf2 = (p_ref[0] + p_ref[1]) * RS_NEI
    t = nf2 * attr_ref[...]
    conv = jnp.dot(t, wl2_ref[...], preferred_element_type=jnp.float32) * RS_D
    ang = jnp.dot(t, wl3_ref[...], preferred_element_type=jnp.float32) * (0.1 * RS_D)
    out_ref[...] = jnp.cos(ang) * nsc_ref[...] + jnp.sin(ang) * conv


def _final(partials, nsc, attr, W_lin2, W_lin3, bn=2000):
    grid = (N_NODES // bn,)
    return pl.pallas_call(
        _final_body,
        grid=grid,
        in_specs=[
            pl.BlockSpec((2, bn, D), lambda i: (0, i, 0)),
            pl.BlockSpec((bn, D), lambda i: (i, 0)),
            pl.BlockSpec((bn, 1), lambda i: (i, 0)),
            pl.BlockSpec((D, D), lambda i: (0, 0)),
            pl.BlockSpec((D, 1), lambda i: (0, 0)),
        ],
        out_specs=pl.BlockSpec((bn, D), lambda i: (i, 0)),
        out_shape=jax.ShapeDtypeStruct((N_NODES, D), jnp.float32),
    )(partials, nsc, attr, W_lin2, W_lin3)


def kernel(node_input, node_attr, edge_src, edge_dst, edge_attr, edge_scalars,
           W_sc, W_lin1, W_lin2, W_lin3, fc_w0, fc_w1):
    src = edge_src.astype(jnp.int32)
    dst = edge_dst.astype(jnp.int32)
    nsc, nf = _node_mm(node_input, node_attr, W_sc, W_lin1)
    coeff = _edge_mlp(edge_scalars, edge_attr, fc_w0, fc_w1)
    partials = _sc_gather_scatter(nf, coeff, src, dst)
    partials = partials.reshape(NC, N_PAD, D)
    return _final(partials, nsc, node_attr, W_lin2, W_lin3)


# speedup vs baseline: 3.2612x; 1.0130x over previous
import functools

import jax
import jax.numpy as jnp
from jax import lax
from jax.experimental import pallas as pl
from jax.experimental.pallas import tpu as pltpu
from jax.experimental.pallas import tpu_sc as plsc

N_NODES = 10000
N_EDGES = 320000
D = 128
FC_IN = 16
FC_HID = 64
SILU_GAIN = 1.6789
RS_D = 1.0 / (D ** 0.5)
RS_IN = 1.0 / (FC_IN ** 0.5)
RS_HID = 1.0 / (FC_HID ** 0.5)
RS_NEI = 1.0 / (32.0 ** 0.5)

NC = 2
NS = 16
NW = NC * NS
NSLICE = 2
E_SL = N_EDGES // NSLICE
EPW = E_SL // NW
K = 40
NCHUNK = EPW // K
N_PAD = 10240
ROWS_PER_SUB = N_PAD // NS


def _node_mm_body(x_ref, attr_ref, wsc_ref, wl1_ref, nsc_ref, nf_ref):
    t = x_ref[...] * attr_ref[...]
    nsc_ref[...] = jnp.dot(t, wsc_ref[...], preferred_element_type=jnp.float32) * RS_D
    nf_ref[...] = jnp.dot(t, wl1_ref[...], preferred_element_type=jnp.float32) * RS_D


def _node_mm(x, attr, W_sc, W_lin1, bn=2000):
    grid = (N_NODES // bn,)
    return pl.pallas_call(
        _node_mm_body,
        grid=grid,
        in_specs=[
            pl.BlockSpec((bn, D), lambda i: (i, 0)),
            pl.BlockSpec((bn, 1), lambda i: (i, 0)),
            pl.BlockSpec((D, D), lambda i: (0, 0)),
            pl.BlockSpec((D, D), lambda i: (0, 0)),
        ],
        out_specs=[
            pl.BlockSpec((bn, D), lambda i: (i, 0)),
            pl.BlockSpec((bn, D), lambda i: (i, 0)),
        ],
        out_shape=[
            jax.ShapeDtypeStruct((N_NODES, D), jnp.float32),
            jax.ShapeDtypeStruct((N_NODES, D), jnp.float32),
        ],
    )(x, attr, W_sc, W_lin1)


def _edge_mlp_body(es_ref, ea_ref, w0_ref, w1_ref, coeff_ref):
    h = jnp.dot(es_ref[...], w0_ref[...], preferred_element_type=jnp.float32) * RS_IN
    h = jax.nn.silu(h) * SILU_GAIN
    w = jnp.dot(h, w1_ref[...], preferred_element_type=jnp.float32) * RS_HID
    coeff_ref[...] = w * ea_ref[...]


def _edge_mlp(edge_scalars, edge_attr, fc_w0, fc_w1, sl, be=8000):
    grid = (E_SL // be,)
    blk0 = sl * (E_SL // be)
    return pl.pallas_call(
        _edge_mlp_body,
        grid=grid,
        in_specs=[
            pl.BlockSpec((be, FC_IN), lambda i: (blk0 + i, 0)),
            pl.BlockSpec((be, 1), lambda i: (blk0 + i, 0)),
            pl.BlockSpec((FC_IN, FC_HID), lambda i: (0, 0)),
            pl.BlockSpec((FC_HID, D), lambda i: (0, 0)),
        ],
        out_specs=pl.BlockSpec((be, D), lambda i: (i, 0)),
        out_shape=jax.ShapeDtypeStruct((E_SL, D), jnp.float32),
    )(edge_scalars, edge_attr, fc_w0, fc_w1)


def _sc_gather_scatter(nf, coeff, src, dst, sl):
    mesh = plsc.VectorSubcoreMesh(core_axis_name="c", subcore_axis_name="s")

    @functools.partial(
        pl.kernel,
        mesh=mesh,
        out_type=jax.ShapeDtypeStruct((NC * N_PAD, D), jnp.float32),
        scratch_types=[
            pltpu.VMEM((K,), jnp.int32),
            pltpu.VMEM((K,), jnp.int32),
            pltpu.VMEM((K, D), jnp.float32),
            pltpu.VMEM((K, D), jnp.float32),
            pltpu.VMEM((K,), jnp.int32),
            pltpu.VMEM((K,), jnp.int32),
            pltpu.VMEM((K, D), jnp.float32),
            pltpu.VMEM((K, D), jnp.float32),
            pltpu.VMEM_SHARED((N_PAD, D), jnp.float32),
            pltpu.SemaphoreType.DMA,
            pltpu.SemaphoreType.DMA,
            pltpu.SemaphoreType.DMA,
            pltpu.SemaphoreType.DMA,
            pltpu.SemaphoreType.DMA,
            pltpu.SemaphoreType.DMA,
        ],
    )
    def body(nf_hbm, coeff_hbm, src_hbm, dst_hbm, out_hbm,
             sidx0, didx0, rows0, coef0, sidx1, didx1, rows1, coef1,
             acc, isem0, isem1, gsem0, gsem1, csem0, csem1):
        cid = lax.axis_index("c")
        sid = lax.axis_index("s")
        wid = sid * NC + cid
        ebase = sl * E_SL + wid * EPW
        cbase = wid * EPW

        slot0 = (sidx0, didx0, rows0, coef0, isem0, gsem0, csem0)
        slot1 = (sidx1, didx1, rows1, coef1, isem1, gsem1, csem1)

        def issue_idx(ci, slot):
            sidx, didx, _, _, isem, _, _ = slot
            base = ebase + ci * K
            pltpu.async_copy(src_hbm.at[pl.ds(base, K)], sidx, isem)
            pltpu.async_copy(dst_hbm.at[pl.ds(base, K)], didx, isem)

        def wait_idx(slot):
            sidx, didx, _, _, isem, _, _ = slot
            pltpu.make_async_copy(src_hbm.at[pl.ds(0, K)], sidx, isem).wait()
            pltpu.make_async_copy(dst_hbm.at[pl.ds(0, K)], didx, isem).wait()

        def issue_gc(ci, slot):
            sidx, _, rows, coef, _, gsem, csem = slot
            pltpu.async_copy(nf_hbm.at[sidx], rows, gsem)
            pltpu.async_copy(coeff_hbm.at[pl.ds(cbase + ci * K, K)], coef, csem)

        def process(slot):
            sidx, didx, rows, coef, _, gsem, csem = slot
            pltpu.make_async_copy(nf_hbm.at[sidx], rows, gsem).wait()
            pltpu.make_async_copy(coeff_hbm.at[pl.ds(0, K)], coef, csem).wait()

            def mul_row(k, c2):
                for j in range(D // 16):
                    s = pl.ds(j * 16, 16)
                    rows[k, s] = rows[k, s] * coef[k, s]
                return c2
            lax.fori_loop(0, K, mul_row, 0)
            pltpu.sync_copy(rows, acc.at[didx], add=True)

        zrow = sid * ROWS_PER_SUB
        rows0[...] = jnp.zeros((K, D), jnp.float32)
        nfull = ROWS_PER_SUB // K
        for t in range(nfull):
            pltpu.sync_copy(rows0, acc.at[pl.ds(zrow + t * K, K)])
        plsc.subcore_barrier()

        issue_idx(0, slot0)
        wait_idx(slot0)
        issue_gc(0, slot0)
        issue_idx(1, slot1)

        def pair(t, carry):
            ca = 2 * t
            cb = 2 * t + 1
            wait_idx(slot1)
            issue_gc(cb, slot1)
            process(slot0)
            issue_idx(ca + 2, slot0)
            process(slot1)
            wait_idx(slot0)
            issue_gc(ca + 2, slot0)
            issue_idx(jnp.minimum(cb + 2, NCHUNK - 1), slot1)
            return carry
        lax.fori_loop(0, (NCHUNK - 1) // 2, pair, 0)

        wait_idx(slot1)
        process(slot0)

        plsc.subcore_barrier()

        orow = cid * N_PAD + zrow
        for t in range(nfull):
            pltpu.sync_copy(acc.at[pl.ds(zrow + t * K, K)], rows0)
            pltpu.sync_copy(rows0, out_hbm.at[pl.ds(orow + t * K, K)])

    return body(nf, coeff, src, dst)


def _final_body(p_ref, q_ref, nsc_ref, attr_ref, wl2_ref, wl3_ref, out_ref):
    nf2 = (p_ref[0] + p_ref[1] + q_ref[0] + q_ref[1]) * RS_NEI
    t = nf2 * attr_ref[...]
    conv = jnp.dot(t, wl2_ref[...], preferred_element_type=jnp.float32) * RS_D
    ang = jnp.dot(t, wl3_ref[...], preferred_element_type=jnp.float32) * (0.1 * RS_D)
    out_ref[...] = jnp.cos(ang) * nsc_ref[...] + jnp.sin(ang) * conv


def _final(p, q, nsc, attr, W_lin2, W_lin3, bn=2000):
    grid = (N_NODES // bn,)
    return pl.pallas_call(
        _final_body,
        grid=grid,
        in_specs=[
            pl.BlockSpec((2, bn, D), lambda i: (0, i, 0)),
            pl.BlockSpec((2, bn, D), lambda i: (0, i, 0)),
            pl.BlockSpec((bn, D), lambda i: (i, 0)),
            pl.BlockSpec((bn, 1), lambda i: (i, 0)),
            pl.BlockSpec((D, D), lambda i: (0, 0)),
            pl.BlockSpec((D, 1), lambda i: (0, 0)),
        ],
        out_specs=pl.BlockSpec((bn, D), lambda i: (i, 0)),
        out_shape=jax.ShapeDtypeStruct((N_NODES, D), jnp.float32),
    )(p, q, nsc, attr, W_lin2, W_lin3)


def kernel(node_input, node_attr, edge_src, edge_dst, edge_attr, edge_scalars,
           W_sc, W_lin1, W_lin2, W_lin3, fc_w0, fc_w1):
    src = edge_src.astype(jnp.int32)
    dst = edge_dst.astype(jnp.int32)
    nsc, nf = _node_mm(node_input, node_attr, W_sc, W_lin1)
    c0 = _edge_mlp(edge_scalars, edge_attr, fc_w0, fc_w1, 0)
    c1 = _edge_mlp(edge_scalars, edge_attr, fc_w0, fc_w1, 1)
    p0 = _sc_gather_scatter(nf, c0, src, dst, 0)
    p1 = _sc_gather_scatter(nf, c1, src, dst, 1)
    p0 = p0.reshape(NC, N_PAD, D)
    p1 = p1.reshape(NC, N_PAD, D)
    return _final(p0, p1, nsc, node_attr, W_lin2, W_lin3)


# speedup vs baseline: 3.5981x; 1.1033x over previous
import functools

import jax
import jax.numpy as jnp
from jax import lax
from jax.experimental import pallas as pl
from jax.experimental.pallas import tpu as pltpu
from jax.experimental.pallas import tpu_sc as plsc

N_NODES = 10000
N_EDGES = 320000
D = 128
FC_IN = 16
FC_HID = 64
SILU_GAIN = 1.6789
RS_D = 1.0 / (D ** 0.5)
RS_IN = 1.0 / (FC_IN ** 0.5)
RS_HID = 1.0 / (FC_HID ** 0.5)
RS_NEI = 1.0 / (32.0 ** 0.5)

NC = 2
NS = 16
NW = NC * NS
SLICES = ((0, 64000), (64000, 256000))
K = 80
N_PAD = 10240
ROWS_PER_SUB = N_PAD // NS


def _node_mm_body(x_ref, attr_ref, wsc_ref, wl1_ref, nsc_ref, nf_ref):
    t = x_ref[...] * attr_ref[...]
    nsc_ref[...] = jnp.dot(t, wsc_ref[...], preferred_element_type=jnp.float32) * RS_D
    nf_ref[...] = jnp.dot(t, wl1_ref[...], preferred_element_type=jnp.float32) * RS_D


def _node_mm(x, attr, W_sc, W_lin1, bn=2000):
    grid = (N_NODES // bn,)
    return pl.pallas_call(
        _node_mm_body,
        grid=grid,
        in_specs=[
            pl.BlockSpec((bn, D), lambda i: (i, 0)),
            pl.BlockSpec((bn, 1), lambda i: (i, 0)),
            pl.BlockSpec((D, D), lambda i: (0, 0)),
            pl.BlockSpec((D, D), lambda i: (0, 0)),
        ],
        out_specs=[
            pl.BlockSpec((bn, D), lambda i: (i, 0)),
            pl.BlockSpec((bn, D), lambda i: (i, 0)),
        ],
        out_shape=[
            jax.ShapeDtypeStruct((N_NODES, D), jnp.float32),
            jax.ShapeDtypeStruct((N_NODES, D), jnp.float32),
        ],
    )(x, attr, W_sc, W_lin1)


def _edge_mlp_body(es_ref, ea_ref, w0_ref, w1_ref, coeff_ref):
    h = jnp.dot(es_ref[...], w0_ref[...], preferred_element_type=jnp.float32) * RS_IN
    h = jax.nn.silu(h) * SILU_GAIN
    w = jnp.dot(h, w1_ref[...], preferred_element_type=jnp.float32) * RS_HID
    coeff_ref[...] = w * ea_ref[...]


def _edge_mlp(edge_scalars, edge_attr, fc_w0, fc_w1, e0, ne, be=8000):
    grid = (ne // be,)
    blk0 = e0 // be
    return pl.pallas_call(
        _edge_mlp_body,
        grid=grid,
        in_specs=[
            pl.BlockSpec((be, FC_IN), lambda i: (blk0 + i, 0)),
            pl.BlockSpec((be, 1), lambda i: (blk0 + i, 0)),
            pl.BlockSpec((FC_IN, FC_HID), lambda i: (0, 0)),
            pl.BlockSpec((FC_HID, D), lambda i: (0, 0)),
        ],
        out_specs=pl.BlockSpec((be, D), lambda i: (i, 0)),
        out_shape=jax.ShapeDtypeStruct((ne, D), jnp.float32),
    )(edge_scalars, edge_attr, fc_w0, fc_w1)


def _sc_gather_scatter(nf, coeff, src, dst, e0, ne):
    epw = ne // NW
    nchunk = epw // K
    mesh = plsc.VectorSubcoreMesh(core_axis_name="c", subcore_axis_name="s")

    @functools.partial(
        pl.kernel,
        mesh=mesh,
        out_type=jax.ShapeDtypeStruct((NC * N_PAD, D), jnp.float32),
        scratch_types=[
            pltpu.VMEM((K,), jnp.int32),
            pltpu.VMEM((K,), jnp.int32),
            pltpu.VMEM((K, D), jnp.float32),
            pltpu.VMEM((K, D), jnp.float32),
            pltpu.VMEM((K,), jnp.int32),
            pltpu.VMEM((K,), jnp.int32),
            pltpu.VMEM((K, D), jnp.float32),
            pltpu.VMEM((K, D), jnp.float32),
            pltpu.VMEM((K,), jnp.int32),
            pltpu.VMEM((K,), jnp.int32),
            pltpu.VMEM_SHARED((N_PAD, D), jnp.float32),
            pltpu.SemaphoreType.DMA,
            pltpu.SemaphoreType.DMA,
            pltpu.SemaphoreType.DMA,
            pltpu.SemaphoreType.DMA,
            pltpu.SemaphoreType.DMA,
            pltpu.SemaphoreType.DMA,
            pltpu.SemaphoreType.DMA,
            pltpu.SemaphoreType.DMA,
        ],
    )
    def body(nf_hbm, coeff_hbm, src_hbm, dst_hbm, out_hbm,
             sidx0, didx0, rows0, coef0, sidx1, didx1, rows1, coef1,
             dsc0, dsc1,
             acc, isem0, isem1, gsem0, gsem1, csem0, csem1, ssem0, ssem1):
        cid = lax.axis_index("c")
        sid = lax.axis_index("s")
        wid = sid * NC + cid
        ebase = e0 + wid * epw
        cbase = wid * epw

        slot0 = (sidx0, didx0, rows0, coef0, dsc0, isem0, gsem0, csem0, ssem0)
        slot1 = (sidx1, didx1, rows1, coef1, dsc1, isem1, gsem1, csem1, ssem1)

        def issue_idx(ci, slot):
            sidx, didx, _, _, _, isem, _, _, _ = slot
            base = ebase + ci * K
            pltpu.async_copy(src_hbm.at[pl.ds(base, K)], sidx, isem)
            pltpu.async_copy(dst_hbm.at[pl.ds(base, K)], didx, isem)

        def wait_idx(slot):
            sidx, didx, _, _, _, isem, _, _, _ = slot
            pltpu.make_async_copy(src_hbm.at[pl.ds(0, K)], sidx, isem).wait()
            pltpu.make_async_copy(dst_hbm.at[pl.ds(0, K)], didx, isem).wait()

        def issue_gc(ci, slot):
            sidx, _, rows, coef, _, _, gsem, csem, _ = slot
            pltpu.async_copy(nf_hbm.at[sidx], rows, gsem)
            pltpu.async_copy(coeff_hbm.at[pl.ds(cbase + ci * K, K)], coef, csem)

        def wait_gc(slot):
            sidx, _, rows, coef, _, _, gsem, csem, _ = slot
            pltpu.make_async_copy(nf_hbm.at[sidx], rows, gsem).wait()
            pltpu.make_async_copy(coeff_hbm.at[pl.ds(0, K)], coef, csem).wait()

        def process(slot):
            _, didx, rows, coef, dsc, _, _, _, ssem = slot
            wait_gc(slot)
            for v in range(K // 16):
                s = pl.ds(v * 16, 16)
                dsc[s] = didx[s]

            def mul_row(k, c2):
                for j in range(D // 16):
                    s = pl.ds(j * 16, 16)
                    rows[k, s] = rows[k, s] * coef[k, s]
                return c2
            lax.fori_loop(0, K, mul_row, 0)
            pltpu.async_copy(rows, acc.at[dsc], add=True, sem=ssem)

        def wait_scatter(slot):
            _, _, rows, _, _, _, _, _, ssem = slot
            pltpu.make_async_copy(rows, acc.at[pl.ds(0, K)], ssem).wait()

        zrow = sid * ROWS_PER_SUB
        rows0[...] = jnp.zeros((K, D), jnp.float32)
        nfull = ROWS_PER_SUB // K
        for t in range(nfull):
            pltpu.sync_copy(rows0, acc.at[pl.ds(zrow + t * K, K)])
        plsc.subcore_barrier()

        issue_idx(0, slot0)
        wait_idx(slot0)
        issue_gc(0, slot0)
        issue_idx(1, slot1)

        def pair(t, carry):
            ca = 2 * t
            cb = 2 * t + 1
            wait_idx(slot1)

            @pl.when(t > 0)
            def _():
                wait_scatter(slot1)
            issue_gc(cb, slot1)
            process(slot0)
            issue_idx(jnp.minimum(ca + 2, nchunk - 1), slot0)
            process(slot1)
            wait_idx(slot0)
            wait_scatter(slot0)
            issue_gc(jnp.minimum(ca + 2, nchunk - 1), slot0)
            issue_idx(jnp.minimum(cb + 2, nchunk - 1), slot1)
            return carry
        lax.fori_loop(0, nchunk // 2, pair, 0)

        wait_idx(slot1)
        if nchunk % 2:
            process(slot0)
            wait_scatter(slot0)
        else:
            wait_gc(slot0)
        wait_scatter(slot1)

        plsc.subcore_barrier()

        orow = cid * N_PAD + zrow
        for t in range(nfull):
            pltpu.sync_copy(acc.at[pl.ds(zrow + t * K, K)], rows0)
            pltpu.sync_copy(rows0, out_hbm.at[pl.ds(orow + t * K, K)])

    return body(nf, coeff, src, dst)


def _final_body(p_ref, q_ref, nsc_ref, attr_ref, wl2_ref, wl3_ref, out_ref):
    nf2 = (p_ref[0] + p_ref[1] + q_ref[0] + q_ref[1]) * RS_NEI
    t = nf2 * attr_ref[...]
    conv = jnp.dot(t, wl2_ref[...], preferred_element_type=jnp.float32) * RS_D
    ang = jnp.dot(t, wl3_ref[...], preferred_element_type=jnp.float32) * (0.1 * RS_D)
    out_ref[...] = jnp.cos(ang) * nsc_ref[...] + jnp.sin(ang) * conv


def _final(p, q, nsc, attr, W_lin2, W_lin3, bn=2000):
    grid = (N_NODES // bn,)
    return pl.pallas_call(
        _final_body,
        grid=grid,
        in_specs=[
            pl.BlockSpec((2, bn, D), lambda i: (0, i, 0)),
            pl.BlockSpec((2, bn, D), lambda i: (0, i, 0)),
            pl.BlockSpec((bn, D), lambda i: (i, 0)),
            pl.BlockSpec((bn, 1), lambda i: (i, 0)),
            pl.BlockSpec((D, D), lambda i: (0, 0)),
            pl.BlockSpec((D, 1), lambda i: (0, 0)),
        ],
        out_specs=pl.BlockSpec((bn, D), lambda i: (i, 0)),
        out_shape=jax.ShapeDtypeStruct((N_NODES, D), jnp.float32),
    )(p, q, nsc, attr, W_lin2, W_lin3)


def kernel(node_input, node_attr, edge_src, edge_dst, edge_attr, edge_scalars,
           W_sc, W_lin1, W_lin2, W_lin3, fc_w0, fc_w1):
    src = edge_src.astype(jnp.int32)
    dst = edge_dst.astype(jnp.int32)
    nsc, nf = _node_mm(node_input, node_attr, W_sc, W_lin1)
    parts = []
    for (e0, ne) in SLICES:
        c = _edge_mlp(edge_scalars, edge_attr, fc_w0, fc_w1, e0, ne)
        p = _sc_gather_scatter(nf, c, src, dst, e0, ne)
        parts.append(p.reshape(NC, N_PAD, D))
    return _final(parts[0], parts[1], nsc, node_attr, W_lin2, W_lin3)


# speedup vs baseline: 3.6056x; 1.0021x over previous
import functools

import jax
import jax.numpy as jnp
from jax import lax
from jax.experimental import pallas as pl
from jax.experimental.pallas import tpu as pltpu
from jax.experimental.pallas import tpu_sc as plsc

N_NODES = 10000
N_EDGES = 320000
D = 128
FC_IN = 16
FC_HID = 64
SILU_GAIN = 1.6789
RS_D = 1.0 / (D ** 0.5)
RS_IN = 1.0 / (FC_IN ** 0.5)
RS_HID = 1.0 / (FC_HID ** 0.5)
RS_NEI = 1.0 / (32.0 ** 0.5)

NC = 2
NS = 16
NW = NC * NS
SLICES = ((0, 64000), (64000, 256000))
K = 80
N_PAD = 10240
ROWS_PER_SUB = N_PAD // NS


def _node_mm_body(x_ref, attr_ref, wsc_ref, wl1_ref, nsc_ref, nf_ref):
    t = x_ref[...] * attr_ref[...]
    nsc_ref[...] = jnp.dot(t, wsc_ref[...], preferred_element_type=jnp.float32) * RS_D
    nf_ref[...] = jnp.dot(t, wl1_ref[...], preferred_element_type=jnp.float32) * RS_D


def _node_mm(x, attr, W_sc, W_lin1, bn=2000):
    grid = (N_NODES // bn,)
    return pl.pallas_call(
        _node_mm_body,
        grid=grid,
        in_specs=[
            pl.BlockSpec((bn, D), lambda i: (i, 0)),
            pl.BlockSpec((bn, 1), lambda i: (i, 0)),
            pl.BlockSpec((D, D), lambda i: (0, 0)),
            pl.BlockSpec((D, D), lambda i: (0, 0)),
        ],
        out_specs=[
            pl.BlockSpec((bn, D), lambda i: (i, 0)),
            pl.BlockSpec((bn, D), lambda i: (i, 0)),
        ],
        out_shape=[
            jax.ShapeDtypeStruct((N_NODES, D), jnp.float32),
            jax.ShapeDtypeStruct((N_NODES, D), jnp.float32),
        ],
    )(x, attr, W_sc, W_lin1)


def _edge_mlp_body(es_ref, ea_ref, w0_ref, w1_ref, coeff_ref):
    h = jnp.dot(es_ref[...], w0_ref[...], preferred_element_type=jnp.float32) * RS_IN
    h = jax.nn.silu(h) * SILU_GAIN
    w = jnp.dot(h, w1_ref[...], preferred_element_type=jnp.float32) * RS_HID
    coeff_ref[...] = w * ea_ref[...]


def _edge_mlp(edge_scalars, edge_attr, fc_w0, fc_w1, e0, ne, be=8000):
    grid = (ne // be,)
    blk0 = e0 // be
    return pl.pallas_call(
        _edge_mlp_body,
        grid=grid,
        in_specs=[
            pl.BlockSpec((be, FC_IN), lambda i: (blk0 + i, 0)),
            pl.BlockSpec((be, 1), lambda i: (blk0 + i, 0)),
            pl.BlockSpec((FC_IN, FC_HID), lambda i: (0, 0)),
            pl.BlockSpec((FC_HID, D), lambda i: (0, 0)),
        ],
        out_specs=pl.BlockSpec((be, D), lambda i: (i, 0)),
        out_shape=jax.ShapeDtypeStruct((ne, D), jnp.float32),
    )(edge_scalars, edge_attr, fc_w0, fc_w1)


def _sc_gather_scatter(nf, coeff, src, dst, e0, ne):
    epw = ne // NW
    nchunk = epw // K
    mesh = plsc.VectorSubcoreMesh(core_axis_name="c", subcore_axis_name="s")

    @functools.partial(
        pl.kernel,
        mesh=mesh,
        out_type=jax.ShapeDtypeStruct((NC * N_PAD, D), jnp.float32),
        scratch_types=[
            pltpu.VMEM((K,), jnp.int32),
            pltpu.VMEM((K,), jnp.int32),
            pltpu.VMEM((K, D), jnp.float32),
            pltpu.VMEM((K, D), jnp.float32),
            pltpu.VMEM((K,), jnp.int32),
            pltpu.VMEM((K,), jnp.int32),
            pltpu.VMEM((K, D), jnp.float32),
            pltpu.VMEM((K, D), jnp.float32),
            pltpu.VMEM((K,), jnp.int32),
            pltpu.VMEM((K,), jnp.int32),
            pltpu.VMEM_SHARED((N_PAD, D), jnp.float32),
            pltpu.SemaphoreType.DMA,
            pltpu.SemaphoreType.DMA,
            pltpu.SemaphoreType.DMA,
            pltpu.SemaphoreType.DMA,
            pltpu.SemaphoreType.DMA,
            pltpu.SemaphoreType.DMA,
            pltpu.SemaphoreType.DMA,
            pltpu.SemaphoreType.DMA,
        ],
    )
    def body(nf_hbm, coeff_hbm, src_hbm, dst_hbm, out_hbm,
             sidx0, didx0, rows0, coef0, sidx1, didx1, rows1, coef1,
             dsc0, dsc1,
             acc, isem0, isem1, gsem0, gsem1, csem0, csem1, ssem0, ssem1):
        cid = lax.axis_index("c")
        sid = lax.axis_index("s")
        wid = sid * NC + cid
        ebase = e0 + wid * epw
        cbase = wid * epw

        slot0 = (sidx0, didx0, rows0, coef0, dsc0, isem0, gsem0, csem0, ssem0)
        slot1 = (sidx1, didx1, rows1, coef1, dsc1, isem1, gsem1, csem1, ssem1)

        def issue_idx(ci, slot):
            sidx, didx, _, _, _, isem, _, _, _ = slot
            base = ebase + ci * K
            pltpu.async_copy(src_hbm.at[pl.ds(base, K)], sidx, isem)
            pltpu.async_copy(dst_hbm.at[pl.ds(base, K)], didx, isem)

        def wait_idx(slot):
            sidx, didx, _, _, _, isem, _, _, _ = slot
            pltpu.make_async_copy(src_hbm.at[pl.ds(0, K)], sidx, isem).wait()
            pltpu.make_async_copy(dst_hbm.at[pl.ds(0, K)], didx, isem).wait()

        def issue_gc(ci, slot):
            sidx, _, rows, coef, _, _, gsem, csem, _ = slot
            pltpu.async_copy(nf_hbm.at[sidx], rows, gsem)
            pltpu.async_copy(coeff_hbm.at[pl.ds(cbase + ci * K, K)], coef, csem)

        def wait_gc(slot):
            sidx, _, rows, coef, _, _, gsem, csem, _ = slot
            pltpu.make_async_copy(nf_hbm.at[sidx], rows, gsem).wait()
            pltpu.make_async_copy(coeff_hbm.at[pl.ds(0, K)], coef, csem).wait()

        def process(slot):
            _, didx, rows, coef, dsc, _, _, _, ssem = slot
            wait_gc(slot)
            for v in range(K // 16):
                s = pl.ds(v * 16, 16)
                dsc[s] = didx[s]

            def mul_row(k, c2):
                for j in range(D // 16):
                    s = pl.ds(j * 16, 16)
                    rows[k, s] = rows[k, s] * coef[k, s]
                return c2
            lax.fori_loop(0, K, mul_row, 0)
            pltpu.async_copy(rows, acc.at[dsc], add=True, sem=ssem)

        def wait_scatter(slot):
            _, _, rows, _, _, _, _, _, ssem = slot
            pltpu.make_async_copy(rows, acc.at[pl.ds(0, K)], ssem).wait()

        zrow = sid * ROWS_PER_SUB
        rows0[...] = jnp.zeros((K, D), jnp.float32)
        nfull = ROWS_PER_SUB // K
        for t in range(nfull):
            pltpu.async_copy(rows0, acc.at[pl.ds(zrow + t * K, K)], gsem0)
        for t in range(nfull):
            pltpu.make_async_copy(rows0, acc.at[pl.ds(zrow, K)], gsem0).wait()
        plsc.subcore_barrier()

        issue_idx(0, slot0)
        wait_idx(slot0)
        issue_gc(0, slot0)
        issue_idx(1, slot1)

        def pair(t, carry):
            ca = 2 * t
            cb = 2 * t + 1
            wait_idx(slot1)

            @pl.when(t > 0)
            def _():
                wait_scatter(slot1)
            issue_gc(cb, slot1)
            process(slot0)
            issue_idx(jnp.minimum(ca + 2, nchunk - 1), slot0)
            process(slot1)
            wait_idx(slot0)
            wait_scatter(slot0)
            issue_gc(jnp.minimum(ca + 2, nchunk - 1), slot0)
            issue_idx(jnp.minimum(cb + 2, nchunk - 1), slot1)
            return carry
        lax.fori_loop(0, nchunk // 2, pair, 0)

        wait_idx(slot1)
        if nchunk % 2:
            process(slot0)
            wait_scatter(slot0)
        else:
            wait_gc(slot0)
        wait_scatter(slot1)

        plsc.subcore_barrier()

        orow = cid * N_PAD + zrow
        for t in range(nfull):
            pltpu.async_copy(acc.at[pl.ds(zrow + t * K, K)],
                             out_hbm.at[pl.ds(orow + t * K, K)], gsem0)
        for t in range(nfull):
            pltpu.make_async_copy(acc.at[pl.ds(zrow, K)],
                                  out_hbm.at[pl.ds(orow, K)], gsem0).wait()

    return body(nf, coeff, src, dst)


def _final_body(p_ref, q_ref, nsc_ref, attr_ref, wl2_ref, wl3_ref, out_ref):
    nf2 = (p_ref[0] + p_ref[1] + q_ref[0] + q_ref[1]) * RS_NEI
    t = nf2 * attr_ref[...]
    conv = jnp.dot(t, wl2_ref[...], preferred_element_type=jnp.float32) * RS_D
    ang = jnp.dot(t, wl3_ref[...], preferred_element_type=jnp.float32) * (0.1 * RS_D)
    out_ref[...] = jnp.cos(ang) * nsc_ref[...] + jnp.sin(ang) * conv


def _final(p, q, nsc, attr, W_lin2, W_lin3, bn=2000):
    grid = (N_NODES // bn,)
    return pl.pallas_call(
        _final_body,
        grid=grid,
        in_specs=[
            pl.BlockSpec((2, bn, D), lambda i: (0, i, 0)),
            pl.BlockSpec((2, bn, D), lambda i: (0, i, 0)),
            pl.BlockSpec((bn, D), lambda i: (i, 0)),
            pl.BlockSpec((bn, 1), lambda i: (i, 0)),
            pl.BlockSpec((D, D), lambda i: (0, 0)),
            pl.BlockSpec((D, 1), lambda i: (0, 0)),
        ],
        out_specs=pl.BlockSpec((bn, D), lambda i: (i, 0)),
        out_shape=jax.ShapeDtypeStruct((N_NODES, D), jnp.float32),
    )(p, q, nsc, attr, W_lin2, W_lin3)


def kernel(node_input, node_attr, edge_src, edge_dst, edge_attr, edge_scalars,
           W_sc, W_lin1, W_lin2, W_lin3, fc_w0, fc_w1):
    src = edge_src.astype(jnp.int32)
    dst = edge_dst.astype(jnp.int32)
    nsc, nf = _node_mm(node_input, node_attr, W_sc, W_lin1)
    parts = []
    for (e0, ne) in SLICES:
        c = _edge_mlp(edge_scalars, edge_attr, fc_w0, fc_w1, e0, ne)
        p = _sc_gather_scatter(nf, c, src, dst, e0, ne)
        parts.append(p.reshape(NC, N_PAD, D))
    return _final(parts[0], parts[1], nsc, node_attr, W_lin2, W_lin3)


# speedup vs baseline: 3.6535x; 1.0133x over previous
import functools

import jax
import jax.numpy as jnp
from jax import lax
from jax.experimental import pallas as pl
from jax.experimental.pallas import tpu as pltpu
from jax.experimental.pallas import tpu_sc as plsc

N_NODES = 10000
N_EDGES = 320000
D = 128
FC_IN = 16
FC_HID = 64
SILU_GAIN = 1.6789
RS_D = 1.0 / (D ** 0.5)
RS_IN = 1.0 / (FC_IN ** 0.5)
RS_HID = 1.0 / (FC_HID ** 0.5)
RS_NEI = 1.0 / (32.0 ** 0.5)

NC = 2
NS = 16
NW = NC * NS
SLICES = ((0, 104960), (104960, 104960), (209920, 110080))
K = 80
N_PAD = 10240
ROWS_PER_SUB = N_PAD // NS


def _node_mm_body(x_ref, attr_ref, wsc_ref, wl1_ref, nsc_ref, nf_ref):
    t = x_ref[...] * attr_ref[...]
    nsc_ref[...] = jnp.dot(t, wsc_ref[...], preferred_element_type=jnp.float32) * RS_D
    nf_ref[...] = jnp.dot(t, wl1_ref[...], preferred_element_type=jnp.float32) * RS_D


def _node_mm(x, attr, W_sc, W_lin1, bn=2000):
    grid = (N_NODES // bn,)
    return pl.pallas_call(
        _node_mm_body,
        grid=grid,
        in_specs=[
            pl.BlockSpec((bn, D), lambda i: (i, 0)),
            pl.BlockSpec((bn, 1), lambda i: (i, 0)),
            pl.BlockSpec((D, D), lambda i: (0, 0)),
            pl.BlockSpec((D, D), lambda i: (0, 0)),
        ],
        out_specs=[
            pl.BlockSpec((bn, D), lambda i: (i, 0)),
            pl.BlockSpec((bn, D), lambda i: (i, 0)),
        ],
        out_shape=[
            jax.ShapeDtypeStruct((N_NODES, D), jnp.float32),
            jax.ShapeDtypeStruct((N_NODES, D), jnp.float32),
        ],
    )(x, attr, W_sc, W_lin1)


def _edge_mlp_body(es_ref, ea_ref, w0_ref, w1_ref, coeff_ref):
    h = jnp.dot(es_ref[...], w0_ref[...], preferred_element_type=jnp.float32) * RS_IN
    h = jax.nn.silu(h) * SILU_GAIN
    w = jnp.dot(h, w1_ref[...], preferred_element_type=jnp.float32) * RS_HID
    coeff_ref[...] = w * ea_ref[...]


def _edge_mlp(edge_scalars, edge_attr, fc_w0, fc_w1, e0, ne, be=2560):
    grid = (ne // be,)
    blk0 = e0 // be
    return pl.pallas_call(
        _edge_mlp_body,
        grid=grid,
        in_specs=[
            pl.BlockSpec((be, FC_IN), lambda i: (blk0 + i, 0)),
            pl.BlockSpec((be, 1), lambda i: (blk0 + i, 0)),
            pl.BlockSpec((FC_IN, FC_HID), lambda i: (0, 0)),
            pl.BlockSpec((FC_HID, D), lambda i: (0, 0)),
        ],
        out_specs=pl.BlockSpec((be, D), lambda i: (i, 0)),
        out_shape=jax.ShapeDtypeStruct((ne, D), jnp.float32),
    )(edge_scalars, edge_attr, fc_w0, fc_w1)


def _sc_gather_scatter(nf, coeff, src, dst, e0, ne):
    epw = ne // NW
    nchunk = epw // K
    mesh = plsc.VectorSubcoreMesh(core_axis_name="c", subcore_axis_name="s")

    @functools.partial(
        pl.kernel,
        mesh=mesh,
        out_type=jax.ShapeDtypeStruct((NC * N_PAD, D), jnp.float32),
        scratch_types=[
            pltpu.VMEM((K,), jnp.int32),
            pltpu.VMEM((K,), jnp.int32),
            pltpu.VMEM((K, D), jnp.float32),
            pltpu.VMEM((K, D), jnp.float32),
            pltpu.VMEM((K,), jnp.int32),
            pltpu.VMEM((K,), jnp.int32),
            pltpu.VMEM((K, D), jnp.float32),
            pltpu.VMEM((K, D), jnp.float32),
            pltpu.VMEM((K,), jnp.int32),
            pltpu.VMEM((K,), jnp.int32),
            pltpu.VMEM_SHARED((N_PAD, D), jnp.float32),
            pltpu.SemaphoreType.DMA,
            pltpu.SemaphoreType.DMA,
            pltpu.SemaphoreType.DMA,
            pltpu.SemaphoreType.DMA,
            pltpu.SemaphoreType.DMA,
            pltpu.SemaphoreType.DMA,
            pltpu.SemaphoreType.DMA,
            pltpu.SemaphoreType.DMA,
        ],
    )
    def body(nf_hbm, coeff_hbm, src_hbm, dst_hbm, out_hbm,
             sidx0, didx0, rows0, coef0, sidx1, didx1, rows1, coef1,
             dsc0, dsc1,
             acc, isem0, isem1, gsem0, gsem1, csem0, csem1, ssem0, ssem1):
        cid = lax.axis_index("c")
        sid = lax.axis_index("s")
        wid = sid * NC + cid
        ebase = e0 + wid * epw
        cbase = wid * epw

        slot0 = (sidx0, didx0, rows0, coef0, dsc0, isem0, gsem0, csem0, ssem0)
        slot1 = (sidx1, didx1, rows1, coef1, dsc1, isem1, gsem1, csem1, ssem1)

        def issue_idx(ci, slot):
            sidx, didx, _, _, _, isem, _, _, _ = slot
            base = ebase + ci * K
            pltpu.async_copy(src_hbm.at[pl.ds(base, K)], sidx, isem)
            pltpu.async_copy(dst_hbm.at[pl.ds(base, K)], didx, isem)

        def wait_idx(slot):
            sidx, didx, _, _, _, isem, _, _, _ = slot
            pltpu.make_async_copy(src_hbm.at[pl.ds(0, K)], sidx, isem).wait()
            pltpu.make_async_copy(dst_hbm.at[pl.ds(0, K)], didx, isem).wait()

        def issue_gc(ci, slot):
            sidx, _, rows, coef, _, _, gsem, csem, _ = slot
            pltpu.async_copy(nf_hbm.at[sidx], rows, gsem)
            pltpu.async_copy(coeff_hbm.at[pl.ds(cbase + ci * K, K)], coef, csem)

        def wait_gc(slot):
            sidx, _, rows, coef, _, _, gsem, csem, _ = slot
            pltpu.make_async_copy(nf_hbm.at[sidx], rows, gsem).wait()
            pltpu.make_async_copy(coeff_hbm.at[pl.ds(0, K)], coef, csem).wait()

        def process(slot):
            _, didx, rows, coef, dsc, _, _, _, ssem = slot
            wait_gc(slot)
            for v in range(K // 16):
                s = pl.ds(v * 16, 16)
                dsc[s] = didx[s]

            def mul_row(k, c2):
                for j in range(D // 16):
                    s = pl.ds(j * 16, 16)
                    rows[k, s] = rows[k, s] * coef[k, s]
                return c2
            lax.fori_loop(0, K, mul_row, 0)
            pltpu.async_copy(rows, acc.at[dsc], add=True, sem=ssem)

        def wait_scatter(slot):
            _, _, rows, _, _, _, _, _, ssem = slot
            pltpu.make_async_copy(rows, acc.at[pl.ds(0, K)], ssem).wait()

        zrow = sid * ROWS_PER_SUB
        rows0[...] = jnp.zeros((K, D), jnp.float32)
        nfull = ROWS_PER_SUB // K
        for t in range(nfull):
            pltpu.async_copy(rows0, acc.at[pl.ds(zrow + t * K, K)], gsem0)
        for t in range(nfull):
            pltpu.make_async_copy(rows0, acc.at[pl.ds(zrow, K)], gsem0).wait()
        plsc.subcore_barrier()

        issue_idx(0, slot0)
        wait_idx(slot0)
        issue_gc(0, slot0)
        issue_idx(1, slot1)

        def pair(t, carry):
            ca = 2 * t
            cb = 2 * t + 1
            wait_idx(slot1)

            @pl.when(t > 0)
            def _():
                wait_scatter(slot1)
            issue_gc(cb, slot1)
            process(slot0)
            issue_idx(jnp.minimum(ca + 2, nchunk - 1), slot0)
            process(slot1)
            wait_idx(slot0)
            wait_scatter(slot0)
            issue_gc(jnp.minimum(ca + 2, nchunk - 1), slot0)
            issue_idx(jnp.minimum(cb + 2, nchunk - 1), slot1)
            return carry
        lax.fori_loop(0, nchunk // 2, pair, 0)

        wait_idx(slot1)
        if nchunk % 2:
            process(slot0)
            wait_scatter(slot0)
        else:
            wait_gc(slot0)
        wait_scatter(slot1)

        plsc.subcore_barrier()

        orow = cid * N_PAD + zrow
        for t in range(nfull):
            pltpu.async_copy(acc.at[pl.ds(zrow + t * K, K)],
                             out_hbm.at[pl.ds(orow + t * K, K)], gsem0)
        for t in range(nfull):
            pltpu.make_async_copy(acc.at[pl.ds(zrow, K)],
                                  out_hbm.at[pl.ds(orow, K)], gsem0).wait()

    return body(nf, coeff, src, dst)


def _final_body(*refs):
    parts = refs[:len(SLICES)]
    nsc_ref, attr_ref, wl2_ref, wl3_ref, out_ref = refs[len(SLICES):]
    nf2 = sum(p[0] + p[1] for p in parts) * RS_NEI
    t = nf2 * attr_ref[...]
    conv = jnp.dot(t, wl2_ref[...], preferred_element_type=jnp.float32) * RS_D
    ang = jnp.dot(t, wl3_ref[...], preferred_element_type=jnp.float32) * (0.1 * RS_D)
    out_ref[...] = jnp.cos(ang) * nsc_ref[...] + jnp.sin(ang) * conv


def _final(parts, nsc, attr, W_lin2, W_lin3, bn=2000):
    grid = (N_NODES // bn,)
    return pl.pallas_call(
        _final_body,
        grid=grid,
        in_specs=[pl.BlockSpec((2, bn, D), lambda i: (0, i, 0))
                  for _ in parts] + [
            pl.BlockSpec((bn, D), lambda i: (i, 0)),
            pl.BlockSpec((bn, 1), lambda i: (i, 0)),
            pl.BlockSpec((D, D), lambda i: (0, 0)),
            pl.BlockSpec((D, 1), lambda i: (0, 0)),
        ],
        out_specs=pl.BlockSpec((bn, D), lambda i: (i, 0)),
        out_shape=jax.ShapeDtypeStruct((N_NODES, D), jnp.float32),
    )(*parts, nsc, attr, W_lin2, W_lin3)


def kernel(node_input, node_attr, edge_src, edge_dst, edge_attr, edge_scalars,
           W_sc, W_lin1, W_lin2, W_lin3, fc_w0, fc_w1):
    src = edge_src.astype(jnp.int32)
    dst = edge_dst.astype(jnp.int32)
    nsc, nf = _node_mm(node_input, node_attr, W_sc, W_lin1)
    parts = []
    for (e0, ne) in SLICES:
        c = _edge_mlp(edge_scalars, edge_attr, fc_w0, fc_w1, e0, ne)
        p = _sc_gather_scatter(nf, c, src, dst, e0, ne)
        parts.append(p.reshape(NC, N_PAD, D))
    return _final(parts, nsc, node_attr, W_lin2, W_lin3)
